# baseline (speedup 1.0000x reference)
import jax
import jax.numpy as jnp
from jax import lax
from jax.experimental import pallas as pl
from jax.experimental.pallas import tpu as pltpu
from jax.experimental.pallas import tpu_sc as plsc

N_KV_HEADS = 8
HEAD_DIM = 128
MAX_SEQ_LEN = 8192
Q_LEN = 16
NROWS = N_KV_HEADS * Q_LEN


def _sc_body(k_ref, v_ref, pos_hbm, kval_hbm, vval_hbm,
             pos_v, idx_v, rows_v, sem):
    wid = lax.axis_index("s") * 2 + lax.axis_index("c")

    def do_cache(cache_ref, val_hbm):
        pltpu.sync_copy(pos_hbm, pos_v)
        lanes = lax.iota(jnp.int32, Q_LEN)
        pos_vec = pos_v[...]
        w = lanes
        for shift in range(1, Q_LEN):
            perm = (lanes + shift) & (Q_LEN - 1)
            p_sh = lax.gather(
                pos_vec, perm[:, None],
                lax.GatherDimensionNumbers(
                    offset_dims=(), collapsed_slice_dims=(0,),
                    start_index_map=(0,)),
                slice_sizes=(1,),
                mode=lax.GatherScatterMode.PROMISE_IN_BOUNDS)
            w = jnp.where(p_sh == pos_vec, jnp.maximum(w, perm), w)
        for h in range(N_KV_HEADS):
            idx_v[pl.ds(h * Q_LEN, Q_LEN)] = w + h * Q_LEN
        pltpu.async_copy(val_hbm.at[idx_v], rows_v, sem).wait()
        for h in range(N_KV_HEADS):
            idx_v[pl.ds(h * Q_LEN, Q_LEN)] = pos_vec + h * MAX_SEQ_LEN
        pltpu.async_copy(rows_v, cache_ref.at[idx_v], sem).wait()

    @pl.when(wid == 0)
    def _():
        do_cache(k_ref, kval_hbm)

    @pl.when(wid == 1)
    def _():
        do_cache(v_ref, vval_hbm)


_sc_update = pl.kernel(
    _sc_body,
    out_type=(),
    mesh=plsc.VectorSubcoreMesh(core_axis_name="c", subcore_axis_name="s"),
    scratch_types=[
        pltpu.VMEM((Q_LEN,), jnp.int32),
        pltpu.VMEM((NROWS,), jnp.int32),
        pltpu.VMEM((NROWS, HEAD_DIM), jnp.float32),
        pltpu.SemaphoreType.DMA,
    ],
)


def kernel(k_cache, v_cache, input_pos, k_val, v_val):
    kc = k_cache.reshape(N_KV_HEADS * MAX_SEQ_LEN, HEAD_DIM)
    vc = v_cache.reshape(N_KV_HEADS * MAX_SEQ_LEN, HEAD_DIM)
    kv = k_val.reshape(NROWS, HEAD_DIM)
    vv = v_val.reshape(NROWS, HEAD_DIM)
    pos = input_pos.astype(jnp.int32)

    k_ref = jax.new_ref(kc)
    v_ref = jax.new_ref(vc)
    _sc_update(k_ref, v_ref, pos, kv, vv)
    ko = k_ref[...]
    vo = v_ref[...]
    return (ko.reshape(k_cache.shape), vo.reshape(v_cache.shape))
